# baseline (speedup 1.0000x reference)
import functools
import math

import jax
import jax.numpy as jnp
from jax import lax
from jax.experimental import pallas as pl
from jax.experimental.pallas import tpu as pltpu
from jax.experimental.pallas import tpu_sc as plsc

_N = 8192
_E = 262144
_B = 4096
_NTILES = 32
_NSUB = 16
_CH = 32


def _proj_body(x_ref, wp_ref, aux_ref, xp_ref, al_ref):
    x = x_ref[...]
    wp = wp_ref[...]
    xp = lax.dot_general(x, wp, (((1,), (1,)), ((), ())),
                         preferred_element_type=jnp.float32)
    xp = xp + aux_ref[0][None, :]
    xp_ref[...] = xp
    als = jnp.sum(xp * aux_ref[1][None, :], axis=-1)
    ald = jnp.sum(xp * aux_ref[2][None, :], axis=-1)
    al_ref[...] = jnp.stack([als, ald])


def _make_proj(din, h, blk=1024):
    return pl.pallas_call(
        _proj_body,
        grid=(_N // blk,),
        in_specs=[
            pl.BlockSpec((blk, din), lambda i: (i, 0)),
            pl.BlockSpec((h, din), lambda i: (0, 0)),
            pl.BlockSpec((3, h), lambda i: (0, 0)),
        ],
        out_specs=[
            pl.BlockSpec((blk, h), lambda i: (i, 0)),
            pl.BlockSpec((2, blk), lambda i: (0, i)),
        ],
        out_shape=[
            jax.ShapeDtypeStruct((_N, h), jnp.float32),
            jax.ShapeDtypeStruct((2, _N), jnp.float32),
        ],
    )


def _cqkv_body(acc_ref, den_ref, wq_ref, wk_ref, wv_ref, q_ref, k_ref, v_ref):
    a = acc_ref[0] + acc_ref[1]
    d = jnp.sum(den_ref[...], axis=0)
    o = jnp.maximum(a / (d[:, None] + 1e-16), 0.0)
    for w_ref, o_ref in ((wq_ref, q_ref), (wk_ref, k_ref), (wv_ref, v_ref)):
        o_ref[...] = lax.dot_general(
            o, w_ref[...], (((1,), (1,)), ((), ())),
            preferred_element_type=jnp.float32).astype(jnp.bfloat16)


def _make_cqkv(h, blk=1024):
    return pl.pallas_call(
        _cqkv_body,
        grid=(_N // blk,),
        in_specs=[
            pl.BlockSpec((2, blk, h), lambda i: (0, i, 0)),
            pl.BlockSpec((_NTILES, blk), lambda i: (0, i)),
            pl.BlockSpec((h, h), lambda i: (0, 0)),
            pl.BlockSpec((h, h), lambda i: (0, 0)),
            pl.BlockSpec((h, h), lambda i: (0, 0)),
        ],
        out_specs=[pl.BlockSpec((blk, h), lambda i: (i, 0))] * 3,
        out_shape=[jax.ShapeDtypeStruct((_N, h), jnp.bfloat16)] * 3,
    )


def _attn_body(scale, q_ref, k_ref, v_ref, o_ref):
    q = q_ref[...]
    k = k_ref[...]
    s = lax.dot_general(q, k, (((1,), (1,)), ((), ())),
                        preferred_element_type=jnp.float32) * scale
    p = jnp.exp(s)
    l = jnp.sum(p, axis=-1, keepdims=True)
    o = lax.dot_general(p.astype(jnp.bfloat16), v_ref[...],
                        (((1,), (0,)), ((), ())),
                        preferred_element_type=jnp.float32)
    o_ref[...] = o / l


def _make_attn(h, bq=512):
    return pl.pallas_call(
        functools.partial(_attn_body, 1.0 / math.sqrt(h)),
        grid=(_N // bq,),
        in_specs=[
            pl.BlockSpec((bq, h), lambda i: (i, 0)),
            pl.BlockSpec((_N, h), lambda i: (0, 0)),
            pl.BlockSpec((_N, h), lambda i: (0, 0)),
        ],
        out_specs=pl.BlockSpec((bq, h), lambda i: (i, 0)),
        out_shape=jax.ShapeDtypeStruct((_N, h), jnp.float32),
    )


def _attn_proj_body(scale, q_ref, k_ref, v_ref, wp_ref, aux_ref,
                    o_ref, xp_ref, al_ref):
    _attn_body(scale, q_ref, k_ref, v_ref, o_ref)
    xp = lax.dot_general(o_ref[...], wp_ref[...], (((1,), (1,)), ((), ())),
                         preferred_element_type=jnp.float32)
    xp = xp + aux_ref[0][None, :]
    xp_ref[...] = xp
    als = jnp.sum(xp * aux_ref[1][None, :], axis=-1)
    ald = jnp.sum(xp * aux_ref[2][None, :], axis=-1)
    al_ref[...] = jnp.stack([als, ald])


def _make_attn_proj(h, h2, bq=512):
    return pl.pallas_call(
        functools.partial(_attn_proj_body, 1.0 / math.sqrt(h)),
        grid=(_N // bq,),
        in_specs=[
            pl.BlockSpec((bq, h), lambda i: (i, 0)),
            pl.BlockSpec((_N, h), lambda i: (0, 0)),
            pl.BlockSpec((_N, h), lambda i: (0, 0)),
            pl.BlockSpec((h2, h), lambda i: (0, 0)),
            pl.BlockSpec((3, h2), lambda i: (0, 0)),
        ],
        out_specs=[
            pl.BlockSpec((bq, h), lambda i: (i, 0)),
            pl.BlockSpec((bq, h2), lambda i: (i, 0)),
            pl.BlockSpec((2, bq), lambda i: (0, i)),
        ],
        out_shape=[
            jax.ShapeDtypeStruct((_N, h), jnp.float32),
            jax.ShapeDtypeStruct((_N, h2), jnp.float32),
            jax.ShapeDtypeStruct((2, _N), jnp.float32),
        ],
    )


def _sc_agg_body(h, ch, src_hbm, dst2_hbm, al_hbm, xp_hbm,
                 acc_out, den_out,
                 als_v, ald_v, den_v, src_all, dst_all, w_v,
                 rows0, rows1, rows2, rows3,
                 acc_s, gsem0, gsem1, gsem2, gsem3, ssem0, ssem1, ssem2, ssem3):
    c = lax.axis_index("c")
    s = lax.axis_index("s")
    wid = c * _NSUB + s
    ept = _E // _NTILES
    base = wid * ept
    nch = ept // ch
    rpt = _N // _NSUB
    zero16 = jnp.zeros((16,), jnp.float32)

    def zrow(i, _):
        for hh in range(h // 16):
            rows0[i, pl.ds(hh * 16, 16)] = zero16
        return 0
    lax.fori_loop(0, ch, zrow, 0)
    for r in range(rpt // ch):
        pltpu.sync_copy(rows0, acc_s.at[pl.ds(s * rpt + r * ch, ch)])

    def zden(i, _):
        den_v[pl.ds(i * 16, 16)] = zero16
        return 0
    lax.fori_loop(0, _N // 16, zden, 0)

    pltpu.sync_copy(src_hbm.at[pl.ds(base, ept)], src_all)
    pltpu.sync_copy(dst2_hbm.at[pl.ds(wid * nch, nch)], dst_all)
    pltpu.sync_copy(al_hbm.at[0], als_v)
    pltpu.sync_copy(al_hbm.at[1], ald_v)
    plsc.subcore_barrier()

    def g_idx(k):
        return src_all.at[pl.ds(k * ch, ch)]

    def wcomp(k):
        def wbody(j, _):
            isrc = src_all[pl.ds(k * ch + j * 16, 16)]
            idst = dst_all[k, pl.ds(j * 16, 16)]
            a = plsc.load_gather(als_v, [isrc]) + plsc.load_gather(ald_v, [idst])
            a = jnp.where(a >= 0, a, 0.2 * a)
            w = jnp.exp(a)
            w_v[pl.ds(j * 16, 16)] = w
            plsc.addupdate_scatter(den_v, [idst], w)
            return 0
        lax.fori_loop(0, ch // 16, wbody, 0)

    def srow(k, rows):
        def sbody(j, _):
            wvec = w_v[pl.ds(j * 16, 16)]
            for i in range(16):
                e = j * 16 + i
                we = wvec[i]
                for hh in range(h // 16):
                    sl = pl.ds(hh * 16, 16)
                    rows[e, sl] = rows[e, sl] * we
            return 0
        lax.fori_loop(0, ch // 16, sbody, 0)

    rows = (rows0, rows1, rows2, rows3)
    gsem = (gsem0, gsem1, gsem2, gsem3)
    ssem = (ssem0, ssem1, ssem2, ssem3)
    nbuf = 4

    for b in range(nbuf):
        pltpu.async_copy(xp_hbm.at[g_idx(b)], rows[b], gsem[b])

    def pipe(i, _):
        k0 = nbuf * i
        for b in range(nbuf):
            k = k0 + b
            bp = (b + nbuf - 1) % nbuf

            @pl.when(jnp.logical_and(k + nbuf - 1 < nch, k >= 1))
            def _():
                pltpu.make_async_copy(
                    xp_hbm.at[g_idx(0)], rows[bp], ssem[bp]).wait()
                pltpu.async_copy(
                    xp_hbm.at[g_idx(k + nbuf - 1)], rows[bp], gsem[bp])
            wcomp(k)
            pltpu.make_async_copy(xp_hbm.at[g_idx(k)], rows[b], gsem[b]).wait()
            srow(k, rows[b])
            pltpu.async_copy(rows[b], acc_s.at[dst_all.at[k]], ssem[b],
                             add=True)
        return 0
    lax.fori_loop(0, nch // nbuf, pipe, 0)
    for b in range(nbuf):
        pltpu.make_async_copy(xp_hbm.at[g_idx(0)], rows[b], ssem[b]).wait()

    plsc.subcore_barrier()
    pltpu.sync_copy(den_v, den_out.at[wid])
    pltpu.sync_copy(acc_s.at[pl.ds(s * rpt, rpt)],
                    acc_out.at[c, pl.ds(s * rpt, rpt)])


_SC_PARAMS = pltpu.CompilerParams(
    needs_layout_passes=False, use_tc_tiling_on_sc=False)


def _make_agg(h, ch):
    mesh = plsc.VectorSubcoreMesh(core_axis_name="c", subcore_axis_name="s")
    return pl.kernel(
        functools.partial(_sc_agg_body, h, ch),
        mesh=mesh,
        compiler_params=_SC_PARAMS,
        out_type=[
            jax.ShapeDtypeStruct((2, _N, h), jnp.float32),
            jax.ShapeDtypeStruct((_NTILES, _N), jnp.float32),
        ],
        scratch_types=[
            pltpu.VMEM((_N,), jnp.float32),
            pltpu.VMEM((_N,), jnp.float32),
            pltpu.VMEM((_N,), jnp.float32),
            pltpu.VMEM((_E // _NTILES,), jnp.int32),
            pltpu.VMEM((_E // _NTILES // ch, ch), jnp.int32),
            pltpu.VMEM((ch,), jnp.float32),
            pltpu.VMEM((ch, h), jnp.float32),
            pltpu.VMEM((ch, h), jnp.float32),
            pltpu.VMEM((ch, h), jnp.float32),
            pltpu.VMEM((ch, h), jnp.float32),
            pltpu.VMEM_SHARED((_N, h), jnp.float32),
        ] + [pltpu.SemaphoreType.DMA] * 8,
    )


def _sc_link_body(h2, eli_hbm, cur_hbm, wsb_hbm, out_hbm,
                  hidx_v, tidx_v, hrow_v, trow_v, wsb_v, res_v, sem):
    c = lax.axis_index("c")
    s = lax.axis_index("s")
    wid = c * _NSUB + s
    ppt = _B // _NTILES
    base = wid * ppt
    lane = lax.iota(jnp.int32, 16)

    pltpu.sync_copy(wsb_hbm, wsb_v)
    pltpu.sync_copy(eli_hbm.at[0, pl.ds(base, ppt)], hidx_v)
    pltpu.sync_copy(eli_hbm.at[1, pl.ds(base, ppt)], tidx_v)
    pltpu.async_copy(cur_hbm.at[hidx_v], hrow_v, sem).wait()
    pltpu.async_copy(cur_hbm.at[tidx_v], trow_v, sem).wait()

    def pair16(j, _):
        res = jnp.zeros((16,), jnp.float32)
        for i in range(16):
            e = j * 16 + i
            acc = jnp.zeros((16,), jnp.float32)
            for hh in range(h2 // 16):
                sl = pl.ds(hh * 16, 16)
                acc = acc + hrow_v[e, sl] * trow_v[e, sl] * wsb_v[sl]
            z = jnp.sum(acc)
            res = jnp.where(lane == i, z, res)
        z16 = res + wsb_v[pl.ds(h2, 16)][0]
        res_v[pl.ds(j * 16, 16)] = 1.0 / (1.0 + jnp.exp(-z16))
        return 0
    lax.fori_loop(0, ppt // 16, pair16, 0)

    pltpu.sync_copy(res_v, out_hbm.at[pl.ds(base, ppt)])


def _make_link(h2):
    mesh = plsc.VectorSubcoreMesh(core_axis_name="c", subcore_axis_name="s")
    ppt = _B // _NTILES
    return pl.kernel(
        functools.partial(_sc_link_body, h2),
        mesh=mesh,
        compiler_params=_SC_PARAMS,
        out_type=jax.ShapeDtypeStruct((_B,), jnp.float32),
        scratch_types=[
            pltpu.VMEM((ppt,), jnp.int32),
            pltpu.VMEM((ppt,), jnp.int32),
            pltpu.VMEM((ppt, h2), jnp.float32),
            pltpu.VMEM((ppt, h2), jnp.float32),
            pltpu.VMEM((h2 + 16,), jnp.float32),
            pltpu.VMEM((ppt,), jnp.float32),
            pltpu.SemaphoreType.DMA,
        ],
    )


_make_proj = functools.cache(_make_proj)
_make_cqkv = functools.cache(_make_cqkv)
_make_attn = functools.cache(_make_attn)
_make_attn_proj = functools.cache(_make_attn_proj)
_make_agg = functools.cache(_make_agg)
_make_link = functools.cache(_make_link)


def kernel(x, edge_index, edge_label_index, snap, past1, past2,
           Wp1, bp1, as1, ad1, kW1, kb1, q1, Wq1, Wk1, Wv1,
           Wp2, bp2, as2, ad2, kW2, kb2, q2, Wq2, Wk2, Wv2,
           Wpost, bpost):
    src = edge_index[0]

    aux1 = jnp.stack([bp1, as1, ad1])
    aux2 = jnp.stack([bp2, as2, ad2])
    xp1, al1 = _make_proj(128, 128)(x, Wp1, aux1)
    dst2a = edge_index[1].reshape(_E // 32, 32)
    acc1, den1 = _make_agg(128, 32)(src, dst2a, al1, xp1)
    q1m, k1m, v1m = _make_cqkv(128)(acc1, den1, Wq1, Wk1, Wv1)
    cur1, xp2, al2 = _make_attn_proj(128, 64)(q1m, k1m, v1m, Wp2, aux2)

    dst2b = edge_index[1].reshape(_E // 64, 64)
    acc2, den2 = _make_agg(64, 64)(src, dst2b, al2, xp2)
    q2m, k2m, v2m = _make_cqkv(64)(acc2, den2, Wq2, Wk2, Wv2)
    cur2 = _make_attn(64)(q2m, k2m, v2m)

    wsb = jnp.zeros((80,), jnp.float32)
    wsb = wsb.at[:64].set(Wpost[0] + Wpost[1]).at[64].set(bpost[0] + bpost[1])
    h = _make_link(64)(edge_label_index, cur2, wsb)
    return h, cur1, cur2


# speedup vs baseline: 1.0902x; 1.0902x over previous
import functools
import math

import jax
import jax.numpy as jnp
from jax import lax
from jax.experimental import pallas as pl
from jax.experimental.pallas import tpu as pltpu
from jax.experimental.pallas import tpu_sc as plsc

_N = 8192
_E = 262144
_B = 4096
_NTILES = 32
_NSUB = 16
_CH = 32


def _proj_body(x_ref, wp_ref, aux_ref, xp_ref, al_ref):
    x = x_ref[...]
    wp = wp_ref[...]
    xp = lax.dot_general(x, wp, (((1,), (1,)), ((), ())),
                         preferred_element_type=jnp.float32)
    xp = xp + aux_ref[0][None, :]
    xp_ref[...] = xp
    als = jnp.sum(xp * aux_ref[1][None, :], axis=-1)
    ald = jnp.sum(xp * aux_ref[2][None, :], axis=-1)
    al_ref[...] = jnp.stack([als, ald])


def _make_proj(din, h, blk=1024):
    return pl.pallas_call(
        _proj_body,
        grid=(_N // blk,),
        in_specs=[
            pl.BlockSpec((blk, din), lambda i: (i, 0)),
            pl.BlockSpec((h, din), lambda i: (0, 0)),
            pl.BlockSpec((3, h), lambda i: (0, 0)),
        ],
        out_specs=[
            pl.BlockSpec((blk, h), lambda i: (i, 0)),
            pl.BlockSpec((2, blk), lambda i: (0, i)),
        ],
        out_shape=[
            jax.ShapeDtypeStruct((_N, h), jnp.float32),
            jax.ShapeDtypeStruct((2, _N), jnp.float32),
        ],
    )


def _cqkv_body(acc_ref, den_ref, wq_ref, wk_ref, wv_ref, q_ref, k_ref, v_ref):
    a = acc_ref[0] + acc_ref[1]
    d = jnp.sum(den_ref[...], axis=0)
    o = jnp.maximum(a / (d[:, None] + 1e-16), 0.0)
    for w_ref, o_ref in ((wq_ref, q_ref), (wk_ref, k_ref), (wv_ref, v_ref)):
        o_ref[...] = lax.dot_general(
            o, w_ref[...], (((1,), (1,)), ((), ())),
            preferred_element_type=jnp.float32).astype(jnp.bfloat16)


def _make_cqkv(h, blk=1024):
    return pl.pallas_call(
        _cqkv_body,
        grid=(_N // blk,),
        in_specs=[
            pl.BlockSpec((2, blk, h), lambda i: (0, i, 0)),
            pl.BlockSpec((_NTILES, blk), lambda i: (0, i)),
            pl.BlockSpec((h, h), lambda i: (0, 0)),
            pl.BlockSpec((h, h), lambda i: (0, 0)),
            pl.BlockSpec((h, h), lambda i: (0, 0)),
        ],
        out_specs=[pl.BlockSpec((blk, h), lambda i: (i, 0))] * 3,
        out_shape=[jax.ShapeDtypeStruct((_N, h), jnp.bfloat16)] * 3,
    )


def _attn_body(scale, q_ref, k_ref, v_ref, o_ref):
    q = q_ref[...]
    k = k_ref[...]
    s = lax.dot_general(q, k, (((1,), (1,)), ((), ())),
                        preferred_element_type=jnp.float32) * scale
    p = jnp.exp(s)
    l = jnp.sum(p, axis=-1, keepdims=True)
    o = lax.dot_general(p.astype(jnp.bfloat16), v_ref[...],
                        (((1,), (0,)), ((), ())),
                        preferred_element_type=jnp.float32)
    o_ref[...] = o / l


def _make_attn(h, bq=512):
    return pl.pallas_call(
        functools.partial(_attn_body, 1.0 / math.sqrt(h)),
        grid=(_N // bq,),
        in_specs=[
            pl.BlockSpec((bq, h), lambda i: (i, 0)),
            pl.BlockSpec((_N, h), lambda i: (0, 0)),
            pl.BlockSpec((_N, h), lambda i: (0, 0)),
        ],
        out_specs=pl.BlockSpec((bq, h), lambda i: (i, 0)),
        out_shape=jax.ShapeDtypeStruct((_N, h), jnp.float32),
    )


def _attn_proj_body(scale, q_ref, k_ref, v_ref, wp_ref, aux_ref,
                    o_ref, xp_ref, al_ref):
    _attn_body(scale, q_ref, k_ref, v_ref, o_ref)
    xp = lax.dot_general(o_ref[...], wp_ref[...], (((1,), (1,)), ((), ())),
                         preferred_element_type=jnp.float32)
    xp = xp + aux_ref[0][None, :]
    xp_ref[...] = xp
    als = jnp.sum(xp * aux_ref[1][None, :], axis=-1)
    ald = jnp.sum(xp * aux_ref[2][None, :], axis=-1)
    al_ref[...] = jnp.stack([als, ald])


def _make_attn_proj(h, h2, bq=512):
    return pl.pallas_call(
        functools.partial(_attn_proj_body, 1.0 / math.sqrt(h)),
        grid=(_N // bq,),
        in_specs=[
            pl.BlockSpec((bq, h), lambda i: (i, 0)),
            pl.BlockSpec((_N, h), lambda i: (0, 0)),
            pl.BlockSpec((_N, h), lambda i: (0, 0)),
            pl.BlockSpec((h2, h), lambda i: (0, 0)),
            pl.BlockSpec((3, h2), lambda i: (0, 0)),
        ],
        out_specs=[
            pl.BlockSpec((bq, h), lambda i: (i, 0)),
            pl.BlockSpec((bq, h2), lambda i: (i, 0)),
            pl.BlockSpec((2, bq), lambda i: (0, i)),
        ],
        out_shape=[
            jax.ShapeDtypeStruct((_N, h), jnp.float32),
            jax.ShapeDtypeStruct((_N, h2), jnp.float32),
            jax.ShapeDtypeStruct((2, _N), jnp.float32),
        ],
    )


def _sc_agg_body(h, ch, src_hbm, dst2_hbm, al_hbm, xp_hbm,
                 acc_out, den_out,
                 als_v, ald_v, den_v, src_all, dst_all, w_v,
                 rows0, rows1, rows2, rows3,
                 acc_s, gsem0, gsem1, gsem2, gsem3, ssem0, ssem1, ssem2, ssem3):
    c = lax.axis_index("c")
    s = lax.axis_index("s")
    wid = c * _NSUB + s
    ept = _E // _NTILES
    base = wid * ept
    nch = ept // ch
    rpt = _N // _NSUB
    zero16 = jnp.zeros((16,), jnp.float32)

    def zrow(i, _):
        for hh in range(h // 16):
            rows0[i, pl.ds(hh * 16, 16)] = zero16
        return 0
    lax.fori_loop(0, ch, zrow, 0)
    for r in range(rpt // ch):
        pltpu.sync_copy(rows0, acc_s.at[pl.ds(s * rpt + r * ch, ch)])

    def zden(i, _):
        den_v[pl.ds(i * 16, 16)] = zero16
        return 0
    lax.fori_loop(0, _N // 16, zden, 0)

    pltpu.sync_copy(src_hbm.at[pl.ds(base, ept)], src_all)
    pltpu.sync_copy(dst2_hbm.at[pl.ds(wid * nch, nch)], dst_all)
    pltpu.sync_copy(al_hbm.at[0], als_v)
    pltpu.sync_copy(al_hbm.at[1], ald_v)
    plsc.subcore_barrier()

    def g_idx(k):
        return src_all.at[pl.ds(k * ch, ch)]

    def wcomp(k):
        def wbody(j, _):
            isrc = src_all[pl.ds(k * ch + j * 16, 16)]
            idst = dst_all[k, pl.ds(j * 16, 16)]
            a = plsc.load_gather(als_v, [isrc]) + plsc.load_gather(ald_v, [idst])
            a = jnp.where(a >= 0, a, 0.2 * a)
            w = jnp.exp(a)
            w_v[pl.ds(j * 16, 16)] = w
            plsc.addupdate_scatter(den_v, [idst], w)
            return 0
        lax.fori_loop(0, ch // 16, wbody, 0)

    def srow(k, rows):
        def sbody(j, _):
            wvec = w_v[pl.ds(j * 16, 16)]
            for i in range(16):
                e = j * 16 + i
                we = wvec[i]
                for hh in range(h // 16):
                    sl = pl.ds(hh * 16, 16)
                    rows[e, sl] = rows[e, sl] * we
            return 0
        lax.fori_loop(0, ch // 16, sbody, 0)

    rows = (rows0, rows1, rows2, rows3)
    gsem = (gsem0, gsem1, gsem2, gsem3)
    ssem = (ssem0, ssem1, ssem2, ssem3)
    nbuf = 4

    for b in range(nbuf):
        pltpu.async_copy(xp_hbm.at[g_idx(b)], rows[b], gsem[b])

    def pipe(i, _):
        k0 = nbuf * i
        for b in range(nbuf):
            k = k0 + b
            bp = (b + nbuf - 1) % nbuf

            @pl.when(jnp.logical_and(k + nbuf - 1 < nch, k >= 1))
            def _():
                pltpu.make_async_copy(
                    xp_hbm.at[g_idx(0)], rows[bp], ssem[bp]).wait()
                pltpu.async_copy(
                    xp_hbm.at[g_idx(k + nbuf - 1)], rows[bp], gsem[bp])
            wcomp(k)
            pltpu.make_async_copy(xp_hbm.at[g_idx(k)], rows[b], gsem[b]).wait()
            srow(k, rows[b])
            pltpu.async_copy(rows[b], acc_s.at[dst_all.at[k]], ssem[b],
                             add=True)
        return 0
    lax.fori_loop(0, nch // nbuf, pipe, 0)
    for b in range(nbuf):
        pltpu.make_async_copy(xp_hbm.at[g_idx(0)], rows[b], ssem[b]).wait()

    plsc.subcore_barrier()
    pltpu.sync_copy(den_v, den_out.at[wid])
    pltpu.sync_copy(acc_s.at[pl.ds(s * rpt, rpt)],
                    acc_out.at[c, pl.ds(s * rpt, rpt)])


_SC_PARAMS = pltpu.CompilerParams(
    needs_layout_passes=False, use_tc_tiling_on_sc=False)


def _make_agg(h, ch):
    mesh = plsc.VectorSubcoreMesh(core_axis_name="c", subcore_axis_name="s")
    return pl.kernel(
        functools.partial(_sc_agg_body, h, ch),
        mesh=mesh,
        compiler_params=_SC_PARAMS,
        out_type=[
            jax.ShapeDtypeStruct((2, _N, h), jnp.float32),
            jax.ShapeDtypeStruct((_NTILES, _N), jnp.float32),
        ],
        scratch_types=[
            pltpu.VMEM((_N,), jnp.float32),
            pltpu.VMEM((_N,), jnp.float32),
            pltpu.VMEM((_N,), jnp.float32),
            pltpu.VMEM((_E // _NTILES,), jnp.int32),
            pltpu.VMEM((_E // _NTILES // ch, ch), jnp.int32),
            pltpu.VMEM((ch,), jnp.float32),
            pltpu.VMEM((ch, h), jnp.float32),
            pltpu.VMEM((ch, h), jnp.float32),
            pltpu.VMEM((ch, h), jnp.float32),
            pltpu.VMEM((ch, h), jnp.float32),
            pltpu.VMEM_SHARED((_N, h), jnp.float32),
        ] + [pltpu.SemaphoreType.DMA] * 8,
    )


def _sc_link_body(h2, eli_hbm, cur_hbm, wsb_hbm, out_hbm,
                  hidx_v, tidx_v, hrow_v, trow_v, wsb_v, res_v, sem):
    c = lax.axis_index("c")
    s = lax.axis_index("s")
    wid = c * _NSUB + s
    ppt = _B // _NTILES
    base = wid * ppt
    lane = lax.iota(jnp.int32, 16)

    pltpu.sync_copy(wsb_hbm, wsb_v)
    pltpu.sync_copy(eli_hbm.at[0, pl.ds(base, ppt)], hidx_v)
    pltpu.sync_copy(eli_hbm.at[1, pl.ds(base, ppt)], tidx_v)
    pltpu.async_copy(cur_hbm.at[hidx_v], hrow_v, sem).wait()
    pltpu.async_copy(cur_hbm.at[tidx_v], trow_v, sem).wait()

    def pair16(j, _):
        res = jnp.zeros((16,), jnp.float32)
        for i in range(16):
            e = j * 16 + i
            acc = jnp.zeros((16,), jnp.float32)
            for hh in range(h2 // 16):
                sl = pl.ds(hh * 16, 16)
                acc = acc + hrow_v[e, sl] * trow_v[e, sl] * wsb_v[sl]
            z = jnp.sum(acc)
            res = jnp.where(lane == i, z, res)
        z16 = res + wsb_v[pl.ds(h2, 16)][0]
        res_v[pl.ds(j * 16, 16)] = 1.0 / (1.0 + jnp.exp(-z16))
        return 0
    lax.fori_loop(0, ppt // 16, pair16, 0)

    pltpu.sync_copy(res_v, out_hbm.at[pl.ds(base, ppt)])


def _make_link(h2):
    mesh = plsc.VectorSubcoreMesh(core_axis_name="c", subcore_axis_name="s")
    ppt = _B // _NTILES
    return pl.kernel(
        functools.partial(_sc_link_body, h2),
        mesh=mesh,
        compiler_params=_SC_PARAMS,
        out_type=jax.ShapeDtypeStruct((_B,), jnp.float32),
        scratch_types=[
            pltpu.VMEM((ppt,), jnp.int32),
            pltpu.VMEM((ppt,), jnp.int32),
            pltpu.VMEM((ppt, h2), jnp.float32),
            pltpu.VMEM((ppt, h2), jnp.float32),
            pltpu.VMEM((h2 + 16,), jnp.float32),
            pltpu.VMEM((ppt,), jnp.float32),
            pltpu.SemaphoreType.DMA,
        ],
    )


_make_proj = functools.cache(_make_proj)
_make_cqkv = functools.cache(_make_cqkv)
_make_attn = functools.cache(_make_attn)
_make_attn_proj = functools.cache(_make_attn_proj)
_make_agg = functools.cache(_make_agg)
_make_link = functools.cache(_make_link)


def kernel(x, edge_index, edge_label_index, snap, past1, past2,
           Wp1, bp1, as1, ad1, kW1, kb1, q1, Wq1, Wk1, Wv1,
           Wp2, bp2, as2, ad2, kW2, kb2, q2, Wq2, Wk2, Wv2,
           Wpost, bpost):
    src = edge_index[0]

    aux1 = jnp.stack([bp1, as1, ad1])
    aux2 = jnp.stack([bp2, as2, ad2])
    xp1, al1 = _make_proj(128, 128)(x, Wp1, aux1)
    dst2a = edge_index[1].reshape(_E // 16, 16)
    acc1, den1 = _make_agg(128, 16)(src, dst2a, al1, xp1)
    q1m, k1m, v1m = _make_cqkv(128)(acc1, den1, Wq1, Wk1, Wv1)
    cur1, xp2, al2 = _make_attn_proj(128, 64)(q1m, k1m, v1m, Wp2, aux2)

    acc2, den2 = _make_agg(64, 16)(src, dst2a, al2, xp2)
    q2m, k2m, v2m = _make_cqkv(64)(acc2, den2, Wq2, Wk2, Wv2)
    cur2 = _make_attn(64)(q2m, k2m, v2m)

    wsb = jnp.zeros((80,), jnp.float32)
    wsb = wsb.at[:64].set(Wpost[0] + Wpost[1]).at[64].set(bpost[0] + bpost[1])
    h = _make_link(64)(edge_label_index, cur2, wsb)
    return h, cur1, cur2


# speedup vs baseline: 1.2600x; 1.1557x over previous
import functools
import math

import jax
import jax.numpy as jnp
from jax import lax
from jax.experimental import pallas as pl
from jax.experimental.pallas import tpu as pltpu
from jax.experimental.pallas import tpu_sc as plsc

_N = 8192
_E = 262144
_B = 4096
_NTILES = 32
_NSUB = 16
_CH = 32


def _proj_body(x_ref, wp_ref, aux_ref, xp_ref, al_ref):
    x = x_ref[...]
    wp = wp_ref[...]
    xp = lax.dot_general(x, wp, (((1,), (1,)), ((), ())),
                         preferred_element_type=jnp.float32)
    xp = xp + aux_ref[0][None, :]
    xp_ref[...] = xp
    als = jnp.sum(xp * aux_ref[1][None, :], axis=-1)
    ald = jnp.sum(xp * aux_ref[2][None, :], axis=-1)
    al_ref[...] = jnp.stack([als, ald])


def _make_proj(din, h, blk=1024):
    return pl.pallas_call(
        _proj_body,
        grid=(_N // blk,),
        in_specs=[
            pl.BlockSpec((blk, din), lambda i: (i, 0)),
            pl.BlockSpec((h, din), lambda i: (0, 0)),
            pl.BlockSpec((3, h), lambda i: (0, 0)),
        ],
        out_specs=[
            pl.BlockSpec((blk, h), lambda i: (i, 0)),
            pl.BlockSpec((2, blk), lambda i: (0, i)),
        ],
        out_shape=[
            jax.ShapeDtypeStruct((_N, h), jnp.float32),
            jax.ShapeDtypeStruct((2, _N), jnp.float32),
        ],
    )


def _cqkv_body(acc_ref, den_ref, wq_ref, wk_ref, wv_ref, q_ref, k_ref, v_ref):
    a = acc_ref[0] + acc_ref[1]
    d = jnp.sum(den_ref[...], axis=0)
    o = jnp.maximum(a / (d[:, None] + 1e-16), 0.0)
    for w_ref, o_ref in ((wq_ref, q_ref), (wk_ref, k_ref), (wv_ref, v_ref)):
        o_ref[...] = lax.dot_general(
            o, w_ref[...], (((1,), (1,)), ((), ())),
            preferred_element_type=jnp.float32).astype(jnp.bfloat16)


def _make_cqkv(h, blk=1024):
    return pl.pallas_call(
        _cqkv_body,
        grid=(_N // blk,),
        in_specs=[
            pl.BlockSpec((2, blk, h), lambda i: (0, i, 0)),
            pl.BlockSpec((_NTILES, blk), lambda i: (0, i)),
            pl.BlockSpec((h, h), lambda i: (0, 0)),
            pl.BlockSpec((h, h), lambda i: (0, 0)),
            pl.BlockSpec((h, h), lambda i: (0, 0)),
        ],
        out_specs=[pl.BlockSpec((blk, h), lambda i: (i, 0))] * 3,
        out_shape=[jax.ShapeDtypeStruct((_N, h), jnp.bfloat16)] * 3,
    )


def _attn_body(scale, q_ref, k_ref, v_ref, o_ref):
    q = q_ref[...]
    k = k_ref[...]
    s = lax.dot_general(q, k, (((1,), (1,)), ((), ())),
                        preferred_element_type=jnp.float32) * scale
    p = jnp.exp(s)
    l = jnp.sum(p, axis=-1, keepdims=True)
    o = lax.dot_general(p.astype(jnp.bfloat16), v_ref[...],
                        (((1,), (0,)), ((), ())),
                        preferred_element_type=jnp.float32)
    o_ref[...] = o / l


def _make_attn(h, bq=512):
    return pl.pallas_call(
        functools.partial(_attn_body, 1.0 / math.sqrt(h)),
        grid=(_N // bq,),
        in_specs=[
            pl.BlockSpec((bq, h), lambda i: (i, 0)),
            pl.BlockSpec((_N, h), lambda i: (0, 0)),
            pl.BlockSpec((_N, h), lambda i: (0, 0)),
        ],
        out_specs=pl.BlockSpec((bq, h), lambda i: (i, 0)),
        out_shape=jax.ShapeDtypeStruct((_N, h), jnp.float32),
    )


def _attn_proj_body(scale, q_ref, k_ref, v_ref, wp_ref, aux_ref,
                    o_ref, xp_ref, al_ref):
    _attn_body(scale, q_ref, k_ref, v_ref, o_ref)
    xp = lax.dot_general(o_ref[...], wp_ref[...], (((1,), (1,)), ((), ())),
                         preferred_element_type=jnp.float32)
    xp = xp + aux_ref[0][None, :]
    xp_ref[...] = xp
    als = jnp.sum(xp * aux_ref[1][None, :], axis=-1)
    ald = jnp.sum(xp * aux_ref[2][None, :], axis=-1)
    al_ref[...] = jnp.stack([als, ald])


def _fused_body(scale, bq, h2, *refs):
    if h2:
        (acc_ref, den_ref, wq_ref, wk_ref, wv_ref, wp_ref, aux_ref,
         o_ref, xp_ref, al_ref, q_s, k_s, v_s) = refs
    else:
        (acc_ref, den_ref, wq_ref, wk_ref, wv_ref,
         o_ref, q_s, k_s, v_s) = refs
    i = pl.program_id(0)

    @pl.when(i == 0)
    def _():
        a = acc_ref[0] + acc_ref[1]
        d = jnp.sum(den_ref[...], axis=0)
        ov = jnp.maximum(a / (d[:, None] + 1e-16), 0.0)
        for w_ref, dst in ((wq_ref, q_s), (wk_ref, k_s), (wv_ref, v_s)):
            dst[...] = lax.dot_general(
                ov, w_ref[...], (((1,), (1,)), ((), ())),
                preferred_element_type=jnp.float32).astype(jnp.bfloat16)

    q = q_s[pl.ds(i * bq, bq), :]
    s = lax.dot_general(q, k_s[...], (((1,), (1,)), ((), ())),
                        preferred_element_type=jnp.float32) * scale
    p = jnp.exp(s)
    l = jnp.sum(p, axis=-1, keepdims=True)
    o = lax.dot_general(p.astype(jnp.bfloat16), v_s[...],
                        (((1,), (0,)), ((), ())),
                        preferred_element_type=jnp.float32)
    o = o / l
    o_ref[...] = o
    if h2:
        xp = lax.dot_general(o, wp_ref[...], (((1,), (1,)), ((), ())),
                             preferred_element_type=jnp.float32)
        xp = xp + aux_ref[0][None, :]
        xp_ref[...] = xp
        als = jnp.sum(xp * aux_ref[1][None, :], axis=-1)
        ald = jnp.sum(xp * aux_ref[2][None, :], axis=-1)
        al_ref[...] = jnp.stack([als, ald])


def _make_fused(h, h2, bq=512):
    const = lambda i: (0, 0)
    in_specs = [
        pl.BlockSpec((2, _N, h), lambda i: (0, 0, 0)),
        pl.BlockSpec((_NTILES, _N), const),
        pl.BlockSpec((h, h), const),
        pl.BlockSpec((h, h), const),
        pl.BlockSpec((h, h), const),
    ]
    out_specs = [pl.BlockSpec((bq, h), lambda i: (i, 0))]
    out_shape = [jax.ShapeDtypeStruct((_N, h), jnp.float32)]
    if h2:
        in_specs += [pl.BlockSpec((h2, h), const),
                     pl.BlockSpec((3, h2), const)]
        out_specs += [pl.BlockSpec((bq, h2), lambda i: (i, 0)),
                      pl.BlockSpec((2, bq), lambda i: (0, i))]
        out_shape += [jax.ShapeDtypeStruct((_N, h2), jnp.float32),
                      jax.ShapeDtypeStruct((2, _N), jnp.float32)]
    return pl.pallas_call(
        functools.partial(_fused_body, 1.0 / math.sqrt(h), bq, h2),
        grid=(_N // bq,),
        in_specs=in_specs,
        out_specs=out_specs,
        out_shape=out_shape,
        scratch_shapes=[pltpu.VMEM((_N, h), jnp.bfloat16)] * 3,
    )


def _make_attn_proj(h, h2, bq=512):
    return pl.pallas_call(
        functools.partial(_attn_proj_body, 1.0 / math.sqrt(h)),
        grid=(_N // bq,),
        in_specs=[
            pl.BlockSpec((bq, h), lambda i: (i, 0)),
            pl.BlockSpec((_N, h), lambda i: (0, 0)),
            pl.BlockSpec((_N, h), lambda i: (0, 0)),
            pl.BlockSpec((h2, h), lambda i: (0, 0)),
            pl.BlockSpec((3, h2), lambda i: (0, 0)),
        ],
        out_specs=[
            pl.BlockSpec((bq, h), lambda i: (i, 0)),
            pl.BlockSpec((bq, h2), lambda i: (i, 0)),
            pl.BlockSpec((2, bq), lambda i: (0, i)),
        ],
        out_shape=[
            jax.ShapeDtypeStruct((_N, h), jnp.float32),
            jax.ShapeDtypeStruct((_N, h2), jnp.float32),
            jax.ShapeDtypeStruct((2, _N), jnp.float32),
        ],
    )


def _sc_agg_body(h, ch, src_hbm, dst2_hbm, al_hbm, xp_hbm,
                 acc_out, den_out,
                 als_v, ald_v, den_v, src_all, dst_all, w_v,
                 rows0, rows1, rows2, rows3,
                 acc_s, gsem0, gsem1, gsem2, gsem3, ssem0, ssem1, ssem2, ssem3):
    c = lax.axis_index("c")
    s = lax.axis_index("s")
    wid = c * _NSUB + s
    ept = _E // _NTILES
    base = wid * ept
    nch = ept // ch
    rpt = _N // _NSUB
    zero16 = jnp.zeros((16,), jnp.float32)

    def zrow(i, _):
        for hh in range(h // 16):
            rows0[i, pl.ds(hh * 16, 16)] = zero16
        return 0
    lax.fori_loop(0, ch, zrow, 0)
    for r in range(rpt // ch):
        pltpu.sync_copy(rows0, acc_s.at[pl.ds(s * rpt + r * ch, ch)])

    def zden(i, _):
        den_v[pl.ds(i * 16, 16)] = zero16
        return 0
    lax.fori_loop(0, _N // 16, zden, 0)

    pltpu.sync_copy(src_hbm.at[pl.ds(base, ept)], src_all)
    pltpu.sync_copy(dst2_hbm.at[pl.ds(wid * nch, nch)], dst_all)
    pltpu.sync_copy(al_hbm.at[0], als_v)
    pltpu.sync_copy(al_hbm.at[1], ald_v)
    plsc.subcore_barrier()

    def g_idx(k):
        return src_all.at[pl.ds(k * ch, ch)]

    def wcomp(k):
        def wbody(j, _):
            isrc = src_all[pl.ds(k * ch + j * 16, 16)]
            idst = dst_all[k, pl.ds(j * 16, 16)]
            a = plsc.load_gather(als_v, [isrc]) + plsc.load_gather(ald_v, [idst])
            a = jnp.where(a >= 0, a, 0.2 * a)
            w = jnp.exp(a)
            w_v[pl.ds(j * 16, 16)] = w
            plsc.addupdate_scatter(den_v, [idst], w)
            return 0
        lax.fori_loop(0, ch // 16, wbody, 0)

    def srow(k, rows):
        def sbody(j, _):
            wvec = w_v[pl.ds(j * 16, 16)]
            for i in range(16):
                e = j * 16 + i
                we = wvec[i]
                for hh in range(h // 16):
                    sl = pl.ds(hh * 16, 16)
                    rows[e, sl] = rows[e, sl] * we
            return 0
        lax.fori_loop(0, ch // 16, sbody, 0)

    rows = (rows0, rows1, rows2, rows3)
    gsem = (gsem0, gsem1, gsem2, gsem3)
    ssem = (ssem0, ssem1, ssem2, ssem3)
    nbuf = 4

    for b in range(nbuf):
        pltpu.async_copy(xp_hbm.at[g_idx(b)], rows[b], gsem[b])

    def pipe(i, _):
        k0 = nbuf * i
        for b in range(nbuf):
            k = k0 + b
            bp = (b + nbuf - 1) % nbuf

            @pl.when(jnp.logical_and(k + nbuf - 1 < nch, k >= 1))
            def _():
                pltpu.make_async_copy(
                    xp_hbm.at[g_idx(0)], rows[bp], ssem[bp]).wait()
                pltpu.async_copy(
                    xp_hbm.at[g_idx(k + nbuf - 1)], rows[bp], gsem[bp])
            wcomp(k)
            pltpu.make_async_copy(xp_hbm.at[g_idx(k)], rows[b], gsem[b]).wait()
            srow(k, rows[b])
            pltpu.async_copy(rows[b], acc_s.at[dst_all.at[k]], ssem[b],
                             add=True)
        return 0
    lax.fori_loop(0, nch // nbuf, pipe, 0)
    for b in range(nbuf):
        pltpu.make_async_copy(xp_hbm.at[g_idx(0)], rows[b], ssem[b]).wait()

    plsc.subcore_barrier()
    pltpu.sync_copy(den_v, den_out.at[wid])
    pltpu.sync_copy(acc_s.at[pl.ds(s * rpt, rpt)],
                    acc_out.at[c, pl.ds(s * rpt, rpt)])


_SC_PARAMS = pltpu.CompilerParams(
    needs_layout_passes=False, use_tc_tiling_on_sc=False)


def _make_agg(h, ch):
    mesh = plsc.VectorSubcoreMesh(core_axis_name="c", subcore_axis_name="s")
    return pl.kernel(
        functools.partial(_sc_agg_body, h, ch),
        mesh=mesh,
        compiler_params=_SC_PARAMS,
        out_type=[
            jax.ShapeDtypeStruct((2, _N, h), jnp.float32),
            jax.ShapeDtypeStruct((_NTILES, _N), jnp.float32),
        ],
        scratch_types=[
            pltpu.VMEM((_N,), jnp.float32),
            pltpu.VMEM((_N,), jnp.float32),
            pltpu.VMEM((_N,), jnp.float32),
            pltpu.VMEM((_E // _NTILES,), jnp.int32),
            pltpu.VMEM((_E // _NTILES // ch, ch), jnp.int32),
            pltpu.VMEM((ch,), jnp.float32),
            pltpu.VMEM((ch, h), jnp.float32),
            pltpu.VMEM((ch, h), jnp.float32),
            pltpu.VMEM((ch, h), jnp.float32),
            pltpu.VMEM((ch, h), jnp.float32),
            pltpu.VMEM_SHARED((_N, h), jnp.float32),
        ] + [pltpu.SemaphoreType.DMA] * 8,
    )


def _sc_link_body(h2, eli_hbm, cur_hbm, wsb_hbm, out_hbm,
                  hidx_v, tidx_v, hrow_v, trow_v, wsb_v, res_v, sem):
    c = lax.axis_index("c")
    s = lax.axis_index("s")
    wid = c * _NSUB + s
    ppt = _B // _NTILES
    base = wid * ppt
    lane = lax.iota(jnp.int32, 16)

    pltpu.sync_copy(wsb_hbm, wsb_v)
    pltpu.sync_copy(eli_hbm.at[0, pl.ds(base, ppt)], hidx_v)
    pltpu.sync_copy(eli_hbm.at[1, pl.ds(base, ppt)], tidx_v)
    pltpu.async_copy(cur_hbm.at[hidx_v], hrow_v, sem).wait()
    pltpu.async_copy(cur_hbm.at[tidx_v], trow_v, sem).wait()

    def pair16(j, _):
        res = jnp.zeros((16,), jnp.float32)
        for i in range(16):
            e = j * 16 + i
            acc = jnp.zeros((16,), jnp.float32)
            for hh in range(h2 // 16):
                sl = pl.ds(hh * 16, 16)
                acc = acc + hrow_v[e, sl] * trow_v[e, sl] * wsb_v[sl]
            z = jnp.sum(acc)
            res = jnp.where(lane == i, z, res)
        z16 = res + wsb_v[pl.ds(h2, 16)][0]
        res_v[pl.ds(j * 16, 16)] = 1.0 / (1.0 + jnp.exp(-z16))
        return 0
    lax.fori_loop(0, ppt // 16, pair16, 0)

    pltpu.sync_copy(res_v, out_hbm.at[pl.ds(base, ppt)])


def _make_link(h2):
    mesh = plsc.VectorSubcoreMesh(core_axis_name="c", subcore_axis_name="s")
    ppt = _B // _NTILES
    return pl.kernel(
        functools.partial(_sc_link_body, h2),
        mesh=mesh,
        compiler_params=_SC_PARAMS,
        out_type=jax.ShapeDtypeStruct((_B,), jnp.float32),
        scratch_types=[
            pltpu.VMEM((ppt,), jnp.int32),
            pltpu.VMEM((ppt,), jnp.int32),
            pltpu.VMEM((ppt, h2), jnp.float32),
            pltpu.VMEM((ppt, h2), jnp.float32),
            pltpu.VMEM((h2 + 16,), jnp.float32),
            pltpu.VMEM((ppt,), jnp.float32),
            pltpu.SemaphoreType.DMA,
        ],
    )


_make_proj = functools.cache(_make_proj)
_make_cqkv = functools.cache(_make_cqkv)
_make_attn = functools.cache(_make_attn)
_make_attn_proj = functools.cache(_make_attn_proj)
_make_fused = functools.cache(_make_fused)
_make_agg = functools.cache(_make_agg)
_make_link = functools.cache(_make_link)


def kernel(x, edge_index, edge_label_index, snap, past1, past2,
           Wp1, bp1, as1, ad1, kW1, kb1, q1, Wq1, Wk1, Wv1,
           Wp2, bp2, as2, ad2, kW2, kb2, q2, Wq2, Wk2, Wv2,
           Wpost, bpost):
    src = edge_index[0]

    aux1 = jnp.stack([bp1, as1, ad1])
    aux2 = jnp.stack([bp2, as2, ad2])
    xp1, al1 = _make_proj(128, 128)(x, Wp1, aux1)
    dst2a = edge_index[1].reshape(_E // 32, 32)
    acc1, den1 = _make_agg(128, 32)(src, dst2a, al1, xp1)
    cur1, xp2, al2 = _make_fused(128, 64)(acc1, den1, Wq1, Wk1, Wv1,
                                          Wp2, aux2)

    acc2, den2 = _make_agg(64, 32)(src, dst2a, al2, xp2)
    (cur2,) = _make_fused(64, None)(acc2, den2, Wq2, Wk2, Wv2)

    wsb = jnp.zeros((80,), jnp.float32)
    wsb = wsb.at[:64].set(Wpost[0] + Wpost[1]).at[64].set(bpost[0] + bpost[1])
    h = _make_link(64)(edge_label_index, cur2, wsb)
    return h, cur1, cur2


# speedup vs baseline: 1.3101x; 1.0398x over previous
import functools
import math

import jax
import jax.numpy as jnp
from jax import lax
from jax.experimental import pallas as pl
from jax.experimental.pallas import tpu as pltpu
from jax.experimental.pallas import tpu_sc as plsc

_N = 8192
_E = 262144
_B = 4096
_NTILES = 32
_NSUB = 16
_CH = 32


def _proj_body(x_ref, wp_ref, aux_ref, xp_ref, al_ref):
    x = x_ref[...]
    wp = wp_ref[...]
    xp = lax.dot_general(x, wp, (((1,), (1,)), ((), ())),
                         preferred_element_type=jnp.float32)
    xp = xp + aux_ref[0][None, :]
    xp_ref[...] = xp
    als = jnp.sum(xp * aux_ref[1][None, :], axis=-1)
    ald = jnp.sum(xp * aux_ref[2][None, :], axis=-1)
    al_ref[...] = jnp.stack([als, ald])


def _make_proj(din, h, blk=1024):
    return pl.pallas_call(
        _proj_body,
        grid=(_N // blk,),
        in_specs=[
            pl.BlockSpec((blk, din), lambda i: (i, 0)),
            pl.BlockSpec((h, din), lambda i: (0, 0)),
            pl.BlockSpec((3, h), lambda i: (0, 0)),
        ],
        out_specs=[
            pl.BlockSpec((blk, h), lambda i: (i, 0)),
            pl.BlockSpec((2, blk), lambda i: (0, i)),
        ],
        out_shape=[
            jax.ShapeDtypeStruct((_N, h), jnp.float32),
            jax.ShapeDtypeStruct((2, _N), jnp.float32),
        ],
    )


def _cqkv_body(acc_ref, den_ref, wq_ref, wk_ref, wv_ref, q_ref, k_ref, v_ref):
    a = acc_ref[0] + acc_ref[1]
    d = jnp.sum(den_ref[...], axis=0)
    o = jnp.maximum(a / (d[:, None] + 1e-16), 0.0)
    for w_ref, o_ref in ((wq_ref, q_ref), (wk_ref, k_ref), (wv_ref, v_ref)):
        o_ref[...] = lax.dot_general(
            o, w_ref[...], (((1,), (1,)), ((), ())),
            preferred_element_type=jnp.float32).astype(jnp.bfloat16)


def _make_cqkv(h, blk=1024):
    return pl.pallas_call(
        _cqkv_body,
        grid=(_N // blk,),
        in_specs=[
            pl.BlockSpec((2, blk, h), lambda i: (0, i, 0)),
            pl.BlockSpec((_NTILES, blk), lambda i: (0, i)),
            pl.BlockSpec((h, h), lambda i: (0, 0)),
            pl.BlockSpec((h, h), lambda i: (0, 0)),
            pl.BlockSpec((h, h), lambda i: (0, 0)),
        ],
        out_specs=[pl.BlockSpec((blk, h), lambda i: (i, 0))] * 3,
        out_shape=[jax.ShapeDtypeStruct((_N, h), jnp.bfloat16)] * 3,
    )


def _attn_body(scale, q_ref, k_ref, v_ref, o_ref):
    q = q_ref[...]
    k = k_ref[...]
    s = lax.dot_general(q, k, (((1,), (1,)), ((), ())),
                        preferred_element_type=jnp.float32) * scale
    p = jnp.exp(s)
    l = jnp.sum(p, axis=-1, keepdims=True)
    o = lax.dot_general(p.astype(jnp.bfloat16), v_ref[...],
                        (((1,), (0,)), ((), ())),
                        preferred_element_type=jnp.float32)
    o_ref[...] = o / l


def _make_attn(h, bq=512):
    return pl.pallas_call(
        functools.partial(_attn_body, 1.0 / math.sqrt(h)),
        grid=(_N // bq,),
        in_specs=[
            pl.BlockSpec((bq, h), lambda i: (i, 0)),
            pl.BlockSpec((_N, h), lambda i: (0, 0)),
            pl.BlockSpec((_N, h), lambda i: (0, 0)),
        ],
        out_specs=pl.BlockSpec((bq, h), lambda i: (i, 0)),
        out_shape=jax.ShapeDtypeStruct((_N, h), jnp.float32),
    )


def _attn_proj_body(scale, q_ref, k_ref, v_ref, wp_ref, aux_ref,
                    o_ref, xp_ref, al_ref):
    _attn_body(scale, q_ref, k_ref, v_ref, o_ref)
    xp = lax.dot_general(o_ref[...], wp_ref[...], (((1,), (1,)), ((), ())),
                         preferred_element_type=jnp.float32)
    xp = xp + aux_ref[0][None, :]
    xp_ref[...] = xp
    als = jnp.sum(xp * aux_ref[1][None, :], axis=-1)
    ald = jnp.sum(xp * aux_ref[2][None, :], axis=-1)
    al_ref[...] = jnp.stack([als, ald])


def _fused_body(scale, bq, h2, *refs):
    if h2:
        (acc_ref, den_ref, wq_ref, wk_ref, wv_ref, wp_ref, aux_ref,
         o_ref, xp_ref, al_ref, q_s, k_s, v_s) = refs
    else:
        (acc_ref, den_ref, wq_ref, wk_ref, wv_ref,
         o_ref, q_s, k_s, v_s) = refs
    i = pl.program_id(0)

    @pl.when(i == 0)
    def _():
        a = acc_ref[0] + acc_ref[1]
        d = jnp.sum(den_ref[...], axis=0)
        ov = jnp.maximum(a / (d[:, None] + 1e-16), 0.0)
        for w_ref, dst in ((wq_ref, q_s), (wk_ref, k_s), (wv_ref, v_s)):
            dst[...] = lax.dot_general(
                ov, w_ref[...], (((1,), (1,)), ((), ())),
                preferred_element_type=jnp.float32).astype(jnp.bfloat16)

    q = q_s[pl.ds(i * bq, bq), :]
    s = lax.dot_general(q, k_s[...], (((1,), (1,)), ((), ())),
                        preferred_element_type=jnp.float32) * scale
    p = jnp.exp(s)
    l = jnp.sum(p, axis=-1, keepdims=True)
    o = lax.dot_general(p.astype(jnp.bfloat16), v_s[...],
                        (((1,), (0,)), ((), ())),
                        preferred_element_type=jnp.float32)
    o = o / l
    o_ref[...] = o
    if h2:
        xp = lax.dot_general(o, wp_ref[...], (((1,), (1,)), ((), ())),
                             preferred_element_type=jnp.float32)
        xp = xp + aux_ref[0][None, :]
        xp_ref[...] = xp
        als = jnp.sum(xp * aux_ref[1][None, :], axis=-1)
        ald = jnp.sum(xp * aux_ref[2][None, :], axis=-1)
        al_ref[...] = jnp.stack([als, ald])


def _make_fused(h, h2, bq=512):
    const = lambda i: (0, 0)
    in_specs = [
        pl.BlockSpec((2, _N, h), lambda i: (0, 0, 0)),
        pl.BlockSpec((_NTILES, _N), const),
        pl.BlockSpec((h, h), const),
        pl.BlockSpec((h, h), const),
        pl.BlockSpec((h, h), const),
    ]
    out_specs = [pl.BlockSpec((bq, h), lambda i: (i, 0))]
    out_shape = [jax.ShapeDtypeStruct((_N, h), jnp.float32)]
    if h2:
        in_specs += [pl.BlockSpec((h2, h), const),
                     pl.BlockSpec((3, h2), const)]
        out_specs += [pl.BlockSpec((bq, h2), lambda i: (i, 0)),
                      pl.BlockSpec((2, bq), lambda i: (0, i))]
        out_shape += [jax.ShapeDtypeStruct((_N, h2), jnp.float32),
                      jax.ShapeDtypeStruct((2, _N), jnp.float32)]
    return pl.pallas_call(
        functools.partial(_fused_body, 1.0 / math.sqrt(h), bq, h2),
        grid=(_N // bq,),
        in_specs=in_specs,
        out_specs=out_specs,
        out_shape=out_shape,
        scratch_shapes=[pltpu.VMEM((_N, h), jnp.bfloat16)] * 3,
    )


def _make_attn_proj(h, h2, bq=512):
    return pl.pallas_call(
        functools.partial(_attn_proj_body, 1.0 / math.sqrt(h)),
        grid=(_N // bq,),
        in_specs=[
            pl.BlockSpec((bq, h), lambda i: (i, 0)),
            pl.BlockSpec((_N, h), lambda i: (0, 0)),
            pl.BlockSpec((_N, h), lambda i: (0, 0)),
            pl.BlockSpec((h2, h), lambda i: (0, 0)),
            pl.BlockSpec((3, h2), lambda i: (0, 0)),
        ],
        out_specs=[
            pl.BlockSpec((bq, h), lambda i: (i, 0)),
            pl.BlockSpec((bq, h2), lambda i: (i, 0)),
            pl.BlockSpec((2, bq), lambda i: (0, i)),
        ],
        out_shape=[
            jax.ShapeDtypeStruct((_N, h), jnp.float32),
            jax.ShapeDtypeStruct((_N, h2), jnp.float32),
            jax.ShapeDtypeStruct((2, _N), jnp.float32),
        ],
    )


def _sc_agg_body(h, ch, src_hbm, dst2_hbm, al_hbm, xp_hbm,
                 acc_out, den_out,
                 als_v, ald_v, den_v, src_all, dst_all, w_v,
                 rows0, rows1, rows2, rows3,
                 acc_s, gsem0, gsem1, gsem2, gsem3, ssem0, ssem1, ssem2, ssem3):
    c = lax.axis_index("c")
    s = lax.axis_index("s")
    wid = c * _NSUB + s
    ept = _E // _NTILES
    base = wid * ept
    nch = ept // ch
    rpt = _N // _NSUB
    zero16 = jnp.zeros((16,), jnp.float32)

    def zrow(i, _):
        for hh in range(h // 16):
            rows0[i, pl.ds(hh * 16, 16)] = zero16
        return 0
    lax.fori_loop(0, ch, zrow, 0)
    for r in range(rpt // ch):
        pltpu.sync_copy(rows0, acc_s.at[pl.ds(s * rpt + r * ch, ch)])

    def zden(i, _):
        den_v[pl.ds(i * 16, 16)] = zero16
        return 0
    lax.fori_loop(0, _N // 16, zden, 0)

    pltpu.sync_copy(src_hbm.at[pl.ds(base, ept)], src_all)
    pltpu.sync_copy(dst2_hbm.at[pl.ds(wid * nch, nch)], dst_all)
    pltpu.sync_copy(al_hbm.at[0], als_v)
    pltpu.sync_copy(al_hbm.at[1], ald_v)
    plsc.subcore_barrier()

    def g_idx(k):
        return src_all.at[pl.ds(k * ch, ch)]

    def wcomp(k):
        def wbody(j, _):
            isrc = src_all[pl.ds(k * ch + j * 16, 16)]
            idst = dst_all[k, pl.ds(j * 16, 16)]
            a = plsc.load_gather(als_v, [isrc]) + plsc.load_gather(ald_v, [idst])
            a = jnp.where(a >= 0, a, 0.2 * a)
            w = jnp.exp(a)
            w_v[pl.ds(j * 16, 16)] = w
            plsc.addupdate_scatter(den_v, [idst], w)
            return 0
        lax.fori_loop(0, ch // 16, wbody, 0)

    def srow(k, rows):
        def sbody(j, _):
            wvec = w_v[pl.ds(j * 16, 16)]
            for i in range(16):
                e = j * 16 + i
                we = wvec[i]
                for hh in range(h // 16):
                    sl = pl.ds(hh * 16, 16)
                    rows[e, sl] = rows[e, sl] * we
            return 0
        lax.fori_loop(0, ch // 16, sbody, 0)

    rows = (rows0, rows1, rows2, rows3)
    gsem = (gsem0, gsem1, gsem2, gsem3)
    ssem = (ssem0, ssem1, ssem2, ssem3)
    nbuf = 4

    for b in range(nbuf):
        pltpu.async_copy(xp_hbm.at[g_idx(b)], rows[b], gsem[b])

    def pipe(i, _):
        k0 = nbuf * i
        for b in range(nbuf):
            k = k0 + b
            bp = (b + 2) % nbuf

            @pl.when(jnp.logical_and(k + 2 < nch, k >= 2))
            def _():
                pltpu.make_async_copy(
                    xp_hbm.at[g_idx(0)], rows[bp], ssem[bp]).wait()
                pltpu.async_copy(
                    xp_hbm.at[g_idx(k + 2)], rows[bp], gsem[bp])
            wcomp(k)
            pltpu.make_async_copy(xp_hbm.at[g_idx(k)], rows[b], gsem[b]).wait()
            srow(k, rows[b])
            pltpu.async_copy(rows[b], acc_s.at[dst_all.at[k]], ssem[b],
                             add=True)
        return 0
    lax.fori_loop(0, nch // nbuf, pipe, 0)
    for b in range(nbuf):
        pltpu.make_async_copy(xp_hbm.at[g_idx(0)], rows[b], ssem[b]).wait()

    plsc.subcore_barrier()
    pltpu.sync_copy(den_v, den_out.at[wid])
    pltpu.sync_copy(acc_s.at[pl.ds(s * rpt, rpt)],
                    acc_out.at[c, pl.ds(s * rpt, rpt)])


_SC_PARAMS = pltpu.CompilerParams(
    needs_layout_passes=False, use_tc_tiling_on_sc=False)


def _make_agg(h, ch):
    mesh = plsc.VectorSubcoreMesh(core_axis_name="c", subcore_axis_name="s")
    return pl.kernel(
        functools.partial(_sc_agg_body, h, ch),
        mesh=mesh,
        compiler_params=_SC_PARAMS,
        out_type=[
            jax.ShapeDtypeStruct((2, _N, h), jnp.float32),
            jax.ShapeDtypeStruct((_NTILES, _N), jnp.float32),
        ],
        scratch_types=[
            pltpu.VMEM((_N,), jnp.float32),
            pltpu.VMEM((_N,), jnp.float32),
            pltpu.VMEM((_N,), jnp.float32),
            pltpu.VMEM((_E // _NTILES,), jnp.int32),
            pltpu.VMEM((_E // _NTILES // ch, ch), jnp.int32),
            pltpu.VMEM((ch,), jnp.float32),
            pltpu.VMEM((ch, h), jnp.float32),
            pltpu.VMEM((ch, h), jnp.float32),
            pltpu.VMEM((ch, h), jnp.float32),
            pltpu.VMEM((ch, h), jnp.float32),
            pltpu.VMEM_SHARED((_N, h), jnp.float32),
        ] + [pltpu.SemaphoreType.DMA] * 8,
    )


def _sc_link_body(h2, eli_hbm, cur_hbm, wsb_hbm, out_hbm,
                  hidx_v, tidx_v, hrow_v, trow_v, wsb_v, res_v, sem):
    c = lax.axis_index("c")
    s = lax.axis_index("s")
    wid = c * _NSUB + s
    ppt = _B // _NTILES
    base = wid * ppt
    lane = lax.iota(jnp.int32, 16)

    pltpu.sync_copy(wsb_hbm, wsb_v)
    pltpu.sync_copy(eli_hbm.at[0, pl.ds(base, ppt)], hidx_v)
    pltpu.sync_copy(eli_hbm.at[1, pl.ds(base, ppt)], tidx_v)
    pltpu.async_copy(cur_hbm.at[hidx_v], hrow_v, sem).wait()
    pltpu.async_copy(cur_hbm.at[tidx_v], trow_v, sem).wait()

    def pair16(j, _):
        res = jnp.zeros((16,), jnp.float32)
        for i in range(16):
            e = j * 16 + i
            acc = jnp.zeros((16,), jnp.float32)
            for hh in range(h2 // 16):
                sl = pl.ds(hh * 16, 16)
                acc = acc + hrow_v[e, sl] * trow_v[e, sl] * wsb_v[sl]
            z = jnp.sum(acc)
            res = jnp.where(lane == i, z, res)
        z16 = res + wsb_v[pl.ds(h2, 16)][0]
        res_v[pl.ds(j * 16, 16)] = 1.0 / (1.0 + jnp.exp(-z16))
        return 0
    lax.fori_loop(0, ppt // 16, pair16, 0)

    pltpu.sync_copy(res_v, out_hbm.at[pl.ds(base, ppt)])


def _make_link(h2):
    mesh = plsc.VectorSubcoreMesh(core_axis_name="c", subcore_axis_name="s")
    ppt = _B // _NTILES
    return pl.kernel(
        functools.partial(_sc_link_body, h2),
        mesh=mesh,
        compiler_params=_SC_PARAMS,
        out_type=jax.ShapeDtypeStruct((_B,), jnp.float32),
        scratch_types=[
            pltpu.VMEM((ppt,), jnp.int32),
            pltpu.VMEM((ppt,), jnp.int32),
            pltpu.VMEM((ppt, h2), jnp.float32),
            pltpu.VMEM((ppt, h2), jnp.float32),
            pltpu.VMEM((h2 + 16,), jnp.float32),
            pltpu.VMEM((ppt,), jnp.float32),
            pltpu.SemaphoreType.DMA,
        ],
    )


_make_proj = functools.cache(_make_proj)
_make_cqkv = functools.cache(_make_cqkv)
_make_attn = functools.cache(_make_attn)
_make_attn_proj = functools.cache(_make_attn_proj)
_make_fused = functools.cache(_make_fused)
_make_agg = functools.cache(_make_agg)
_make_link = functools.cache(_make_link)


def kernel(x, edge_index, edge_label_index, snap, past1, past2,
           Wp1, bp1, as1, ad1, kW1, kb1, q1, Wq1, Wk1, Wv1,
           Wp2, bp2, as2, ad2, kW2, kb2, q2, Wq2, Wk2, Wv2,
           Wpost, bpost):
    src = edge_index[0]

    aux1 = jnp.stack([bp1, as1, ad1])
    aux2 = jnp.stack([bp2, as2, ad2])
    xp1, al1 = _make_proj(128, 128)(x, Wp1, aux1)
    dst2a = edge_index[1].reshape(_E // 32, 32)
    acc1, den1 = _make_agg(128, 32)(src, dst2a, al1, xp1)
    cur1, xp2, al2 = _make_fused(128, 64)(acc1, den1, Wq1, Wk1, Wv1,
                                          Wp2, aux2)

    acc2, den2 = _make_agg(64, 32)(src, dst2a, al2, xp2)
    (cur2,) = _make_fused(64, None)(acc2, den2, Wq2, Wk2, Wv2)

    wsb = jnp.zeros((80,), jnp.float32)
    wsb = wsb.at[:64].set(Wpost[0] + Wpost[1]).at[64].set(bpost[0] + bpost[1])
    h = _make_link(64)(edge_label_index, cur2, wsb)
    return h, cur1, cur2


# speedup vs baseline: 1.3103x; 1.0002x over previous
import functools
import math

import jax
import jax.numpy as jnp
from jax import lax
from jax.experimental import pallas as pl
from jax.experimental.pallas import tpu as pltpu
from jax.experimental.pallas import tpu_sc as plsc

_N = 8192
_E = 262144
_B = 4096
_NTILES = 32
_NSUB = 16


def _proj_body(x_ref, wp_ref, aux_ref, xp_ref, al_ref):
    x = x_ref[...]
    wp = wp_ref[...]
    xp = lax.dot_general(x, wp, (((1,), (1,)), ((), ())),
                         preferred_element_type=jnp.float32)
    xp = xp + aux_ref[0][None, :]
    xp_ref[...] = xp
    als = jnp.sum(xp * aux_ref[1][None, :], axis=-1)
    ald = jnp.sum(xp * aux_ref[2][None, :], axis=-1)
    al_ref[...] = jnp.stack([als, ald])


def _make_proj(din, h, blk=1024):
    return pl.pallas_call(
        _proj_body,
        grid=(_N // blk,),
        in_specs=[
            pl.BlockSpec((blk, din), lambda i: (i, 0)),
            pl.BlockSpec((h, din), lambda i: (0, 0)),
            pl.BlockSpec((3, h), lambda i: (0, 0)),
        ],
        out_specs=[
            pl.BlockSpec((blk, h), lambda i: (i, 0)),
            pl.BlockSpec((2, blk), lambda i: (0, i)),
        ],
        out_shape=[
            jax.ShapeDtypeStruct((_N, h), jnp.float32),
            jax.ShapeDtypeStruct((2, _N), jnp.float32),
        ],
    )


def _fused_body(scale, bq, h2, *refs):
    if h2:
        (acc_ref, den_ref, wq_ref, wk_ref, wv_ref, wp_ref, aux_ref,
         o_ref, xp_ref, al_ref, q_s, k_s, v_s) = refs
    else:
        (acc_ref, den_ref, wq_ref, wk_ref, wv_ref,
         o_ref, q_s, k_s, v_s) = refs
    i = pl.program_id(0)

    @pl.when(i == 0)
    def _():
        a = acc_ref[0] + acc_ref[1]
        d = jnp.sum(den_ref[...], axis=0)
        ov = jnp.maximum(a / (d[:, None] + 1e-16), 0.0)
        for w_ref, dst in ((wq_ref, q_s), (wk_ref, k_s), (wv_ref, v_s)):
            dst[...] = lax.dot_general(
                ov, w_ref[...], (((1,), (1,)), ((), ())),
                preferred_element_type=jnp.float32).astype(jnp.bfloat16)

    q = q_s[pl.ds(i * bq, bq), :]
    s = lax.dot_general(q, k_s[...], (((1,), (1,)), ((), ())),
                        preferred_element_type=jnp.float32) * scale
    p = jnp.exp(s)
    l = jnp.sum(p, axis=-1, keepdims=True)
    o = lax.dot_general(p.astype(jnp.bfloat16), v_s[...],
                        (((1,), (0,)), ((), ())),
                        preferred_element_type=jnp.float32)
    o = o / l
    o_ref[...] = o
    if h2:
        xp = lax.dot_general(o, wp_ref[...], (((1,), (1,)), ((), ())),
                             preferred_element_type=jnp.float32)
        xp = xp + aux_ref[0][None, :]
        xp_ref[...] = xp
        als = jnp.sum(xp * aux_ref[1][None, :], axis=-1)
        ald = jnp.sum(xp * aux_ref[2][None, :], axis=-1)
        al_ref[...] = jnp.stack([als, ald])


def _make_fused(h, h2, bq=512):
    const = lambda i: (0, 0)
    in_specs = [
        pl.BlockSpec((2, _N, h), lambda i: (0, 0, 0)),
        pl.BlockSpec((_NTILES, _N), const),
        pl.BlockSpec((h, h), const),
        pl.BlockSpec((h, h), const),
        pl.BlockSpec((h, h), const),
    ]
    out_specs = [pl.BlockSpec((bq, h), lambda i: (i, 0))]
    out_shape = [jax.ShapeDtypeStruct((_N, h), jnp.float32)]
    if h2:
        in_specs += [pl.BlockSpec((h2, h), const),
                     pl.BlockSpec((3, h2), const)]
        out_specs += [pl.BlockSpec((bq, h2), lambda i: (i, 0)),
                      pl.BlockSpec((2, bq), lambda i: (0, i))]
        out_shape += [jax.ShapeDtypeStruct((_N, h2), jnp.float32),
                      jax.ShapeDtypeStruct((2, _N), jnp.float32)]
    return pl.pallas_call(
        functools.partial(_fused_body, 1.0 / math.sqrt(h), bq, h2),
        grid=(_N // bq,),
        in_specs=in_specs,
        out_specs=out_specs,
        out_shape=out_shape,
        scratch_shapes=[pltpu.VMEM((_N, h), jnp.bfloat16)] * 3,
    )


def _sc_agg_body(h, ch, src_hbm, dst2_hbm, al_hbm, xp_hbm,
                 acc_out, den_out,
                 als_v, ald_v, den_v, src_all, dst_all, w_v,
                 rows0, rows1, rows2, rows3,
                 acc_s, gsem0, gsem1, gsem2, gsem3, ssem0, ssem1, ssem2, ssem3):
    c = lax.axis_index("c")
    s = lax.axis_index("s")
    wid = c * _NSUB + s
    ept = _E // _NTILES
    base = wid * ept
    nch = ept // ch
    rpt = _N // _NSUB
    zero16 = jnp.zeros((16,), jnp.float32)

    def zrow(i, _):
        for hh in range(h // 16):
            rows0[i, pl.ds(hh * 16, 16)] = zero16
        return 0
    lax.fori_loop(0, ch, zrow, 0)
    for r in range(rpt // ch):
        pltpu.sync_copy(rows0, acc_s.at[pl.ds(s * rpt + r * ch, ch)])

    def zden(i, _):
        den_v[pl.ds(i * 16, 16)] = zero16
        return 0
    lax.fori_loop(0, _N // 16, zden, 0)

    pltpu.sync_copy(src_hbm.at[pl.ds(base, ept)], src_all)
    pltpu.sync_copy(dst2_hbm.at[pl.ds(wid * nch, nch)], dst_all)
    pltpu.sync_copy(al_hbm.at[0], als_v)
    pltpu.sync_copy(al_hbm.at[1], ald_v)
    plsc.subcore_barrier()

    def g_idx(k):
        return src_all.at[pl.ds(k * ch, ch)]

    def wcomp(k):
        def wbody(j, _):
            isrc = src_all[pl.ds(k * ch + j * 16, 16)]
            idst = dst_all[k, pl.ds(j * 16, 16)]
            a = plsc.load_gather(als_v, [isrc]) + plsc.load_gather(ald_v, [idst])
            a = jnp.where(a >= 0, a, 0.2 * a)
            w = jnp.exp(a)
            w_v[pl.ds(j * 16, 16)] = w
            plsc.addupdate_scatter(den_v, [idst], w)
            return 0
        lax.fori_loop(0, ch // 16, wbody, 0)

    def srow(k, rows):
        def sbody(j, _):
            wvec = w_v[pl.ds(j * 16, 16)]
            for i in range(16):
                e = j * 16 + i
                we = wvec[i]
                for hh in range(h // 16):
                    sl = pl.ds(hh * 16, 16)
                    rows[e, sl] = rows[e, sl] * we
            return 0
        lax.fori_loop(0, ch // 16, sbody, 0)

    rows = (rows0, rows1, rows2, rows3)
    gsem = (gsem0, gsem1, gsem2, gsem3)
    ssem = (ssem0, ssem1, ssem2, ssem3)
    nbuf = 4

    for b in range(nbuf):
        pltpu.async_copy(xp_hbm.at[g_idx(b)], rows[b], gsem[b])

    def pipe(i, _):
        k0 = nbuf * i
        for b in range(nbuf):
            k = k0 + b
            bp = (b + 2) % nbuf

            @pl.when(jnp.logical_and(k + 2 < nch, k >= 2))
            def _():
                pltpu.make_async_copy(
                    xp_hbm.at[g_idx(0)], rows[bp], ssem[bp]).wait()
                pltpu.async_copy(
                    xp_hbm.at[g_idx(k + 2)], rows[bp], gsem[bp])
            wcomp(k)
            pltpu.make_async_copy(xp_hbm.at[g_idx(k)], rows[b], gsem[b]).wait()
            srow(k, rows[b])
            pltpu.async_copy(rows[b], acc_s.at[dst_all.at[k]], ssem[b],
                             add=True)
        return 0
    lax.fori_loop(0, nch // nbuf, pipe, 0)
    for b in range(nbuf):
        pltpu.make_async_copy(xp_hbm.at[g_idx(0)], rows[b], ssem[b]).wait()

    plsc.subcore_barrier()
    pltpu.sync_copy(den_v, den_out.at[wid])
    pltpu.sync_copy(acc_s.at[pl.ds(s * rpt, rpt)],
                    acc_out.at[c, pl.ds(s * rpt, rpt)])


_SC_PARAMS = pltpu.CompilerParams(
    needs_layout_passes=False, use_tc_tiling_on_sc=False)


def _make_agg(h, ch):
    mesh = plsc.VectorSubcoreMesh(core_axis_name="c", subcore_axis_name="s")
    return pl.kernel(
        functools.partial(_sc_agg_body, h, ch),
        mesh=mesh,
        compiler_params=_SC_PARAMS,
        out_type=[
            jax.ShapeDtypeStruct((2, _N, h), jnp.float32),
            jax.ShapeDtypeStruct((_NTILES, _N), jnp.float32),
        ],
        scratch_types=[
            pltpu.VMEM((_N,), jnp.float32),
            pltpu.VMEM((_N,), jnp.float32),
            pltpu.VMEM((_N,), jnp.float32),
            pltpu.VMEM((_E // _NTILES,), jnp.int32),
            pltpu.VMEM((_E // _NTILES // ch, ch), jnp.int32),
            pltpu.VMEM((ch,), jnp.float32),
            pltpu.VMEM((ch, h), jnp.float32),
            pltpu.VMEM((ch, h), jnp.float32),
            pltpu.VMEM((ch, h), jnp.float32),
            pltpu.VMEM((ch, h), jnp.float32),
            pltpu.VMEM_SHARED((_N, h), jnp.float32),
        ] + [pltpu.SemaphoreType.DMA] * 8,
    )


def _sc_link_body(h2, eli_hbm, cur_hbm, wsb_hbm, out_hbm,
                  hidx_v, tidx_v, hrow_v, trow_v, wsb_v, res_v, sem):
    c = lax.axis_index("c")
    s = lax.axis_index("s")
    wid = c * _NSUB + s
    ppt = _B // _NTILES
    base = wid * ppt
    lane = lax.iota(jnp.int32, 16)

    pltpu.sync_copy(wsb_hbm, wsb_v)
    pltpu.sync_copy(eli_hbm.at[0, pl.ds(base, ppt)], hidx_v)
    pltpu.sync_copy(eli_hbm.at[1, pl.ds(base, ppt)], tidx_v)
    pltpu.async_copy(cur_hbm.at[hidx_v], hrow_v, sem).wait()
    pltpu.async_copy(cur_hbm.at[tidx_v], trow_v, sem).wait()

    def pair16(j, _):
        res = jnp.zeros((16,), jnp.float32)
        for i in range(16):
            e = j * 16 + i
            acc = jnp.zeros((16,), jnp.float32)
            for hh in range(h2 // 16):
                sl = pl.ds(hh * 16, 16)
                acc = acc + hrow_v[e, sl] * trow_v[e, sl] * wsb_v[sl]
            z = jnp.sum(acc)
            res = jnp.where(lane == i, z, res)
        z16 = res + wsb_v[pl.ds(h2, 16)][0]
        res_v[pl.ds(j * 16, 16)] = 1.0 / (1.0 + jnp.exp(-z16))
        return 0
    lax.fori_loop(0, ppt // 16, pair16, 0)

    pltpu.sync_copy(res_v, out_hbm.at[pl.ds(base, ppt)])


def _make_link(h2):
    mesh = plsc.VectorSubcoreMesh(core_axis_name="c", subcore_axis_name="s")
    ppt = _B // _NTILES
    return pl.kernel(
        functools.partial(_sc_link_body, h2),
        mesh=mesh,
        compiler_params=_SC_PARAMS,
        out_type=jax.ShapeDtypeStruct((_B,), jnp.float32),
        scratch_types=[
            pltpu.VMEM((ppt,), jnp.int32),
            pltpu.VMEM((ppt,), jnp.int32),
            pltpu.VMEM((ppt, h2), jnp.float32),
            pltpu.VMEM((ppt, h2), jnp.float32),
            pltpu.VMEM((h2 + 16,), jnp.float32),
            pltpu.VMEM((ppt,), jnp.float32),
            pltpu.SemaphoreType.DMA,
        ],
    )


_make_proj = functools.cache(_make_proj)
_make_fused = functools.cache(_make_fused)
_make_agg = functools.cache(_make_agg)
_make_link = functools.cache(_make_link)


def kernel(x, edge_index, edge_label_index, snap, past1, past2,
           Wp1, bp1, as1, ad1, kW1, kb1, q1, Wq1, Wk1, Wv1,
           Wp2, bp2, as2, ad2, kW2, kb2, q2, Wq2, Wk2, Wv2,
           Wpost, bpost):
    src = edge_index[0]

    aux1 = jnp.stack([bp1, as1, ad1])
    aux2 = jnp.stack([bp2, as2, ad2])
    xp1, al1 = _make_proj(128, 128)(x, Wp1, aux1)
    dst2a = edge_index[1].reshape(_E // 32, 32)
    acc1, den1 = _make_agg(128, 32)(src, dst2a, al1, xp1)
    cur1, xp2, al2 = _make_fused(128, 64)(acc1, den1, Wq1, Wk1, Wv1,
                                          Wp2, aux2)

    acc2, den2 = _make_agg(64, 32)(src, dst2a, al2, xp2)
    (cur2,) = _make_fused(64, None)(acc2, den2, Wq2, Wk2, Wv2)

    wsb = jnp.zeros((80,), jnp.float32)
    wsb = wsb.at[:64].set(Wpost[0] + Wpost[1]).at[64].set(bpost[0] + bpost[1])
    h = _make_link(64)(edge_label_index, cur2, wsb)
    return h, cur1, cur2
